# baseline (speedup 1.0000x reference)
import functools

import jax
import jax.numpy as jnp
from jax.experimental import pallas as pl
from jax.experimental.pallas import tpu as pltpu

_F32 = jnp.float32
_CC = 0.25


def _mm(a, b):
    return jax.lax.dot_general(a, b, (((1,), (0,)), ((), ())),
                               preferred_element_type=_F32)


_P = 4


def _prevp(a, s):
    za = jnp.zeros_like(a[..., :1])
    parts = []
    for g in range(0, a.shape[-1], s):
        parts += [za, a[..., g:g + s - 1]]
    return jnp.concatenate(parts, axis=-1)


def _nextp(a, s):
    za = jnp.zeros_like(a[..., :1])
    parts = []
    for g in range(0, a.shape[-1], s):
        parts += [a[..., g + 1:g + s], za]
    return jnp.concatenate(parts, axis=-1)


def _res_block(x, w1, w2, pv, nx):
    a = jax.nn.relu(x)
    r = pv(_mm(w1[0], a)) + _mm(w1[1], a) + nx(_mm(w1[2], a))
    r = jax.nn.relu(r)
    return x + _mm(w2, r)


def _vqvae_body(x4_ref, w1_ref, b1_ref, w2_ref, b2_ref, w3_ref, b3_ref,
                er0w1_ref, er0w2_ref, er1w1_ref, er1w2_ref,
                wpre_ref, bpre_ref, emb_ref,
                dw1_ref, db1_ref, dr0w1_ref, dr0w2_ref, dr1w1_ref, dr1w2_ref,
                t1_ref, bt1_ref, w2c_ref, bt2_ref,
                r4_ref, idx_ref, hist_ref, sq_ref):
    x4 = x4_ref[0]
    seg = x4.shape[-1] // _P
    pv = lambda a: _prevp(a, seg)
    nx = lambda a: _nextp(a, seg)

    means, stds, xparts = [], [], []
    for g in range(_P):
        xg = x4[:, g * seg:(g + 1) * seg]
        mg = jnp.mean(xg)
        sg = jnp.sqrt(jnp.mean((xg - mg) ** 2) + 1e-5)
        means.append(mg)
        stds.append(sg)
        xparts.append((xg - mg) / sg)
    xn = jnp.concatenate(xparts, axis=-1)

    s_e = jnp.concatenate([pv(xn[3:4]), xn[0:3]], axis=0)
    s_o = jnp.concatenate([xn[1:4], nx(xn[0:1])], axis=0)
    b1 = b1_ref[...]
    y1e = jax.nn.relu(_mm(w1_ref[...], s_e) + b1)
    y1o = jax.nn.relu(_mm(w1_ref[...], s_o) + b1)

    w2 = w2_ref[...]
    y2 = (_mm(w2[0], pv(y1o)) + _mm(w2[1], y1e) +
          _mm(w2[2], y1o) + _mm(w2[3], nx(y1e)) + b2_ref[...])
    y2 = jax.nn.relu(y2)

    w3 = w3_ref[...]
    h = (_mm(w3[0], pv(y2)) + _mm(w3[1], y2) + _mm(w3[2], nx(y2))
         + b3_ref[...])

    h = _res_block(h, er0w1_ref[...], er0w2_ref[...], pv, nx)
    h = _res_block(h, er1w1_ref[...], er1w2_ref[...], pv, nx)
    h = jax.nn.relu(h)

    z = _mm(wpre_ref[...], h) + bpre_ref[...]

    emb = emb_ref[...]
    kk = emb.shape[0]
    lf = z.shape[1]
    esq = jnp.sum(emb * emb, axis=1, keepdims=True)
    zsq = jnp.sum(z * z, axis=0, keepdims=True)
    scores = _mm(emb, z)
    d2 = zsq - 2.0 * scores + esq
    dmin = jnp.min(d2, axis=0, keepdims=True)
    kio = jax.lax.broadcasted_iota(jnp.int32, (kk, lf), 0)
    idx = jnp.min(jnp.where(d2 == dmin, kio, kk), axis=0)
    eq = (kio == idx[None, :]).astype(_F32)
    q = jax.lax.dot_general(emb, eq, (((0,), (0,)), ((), ())),
                            preferred_element_type=_F32)
    hist = jnp.sum(eq, axis=1, keepdims=True)
    sq = jnp.sum((q - z) ** 2)

    dw1 = dw1_ref[...]
    h = (_mm(dw1[0], pv(q)) + _mm(dw1[1], q) + _mm(dw1[2], nx(q))
         + db1_ref[...])

    h = _res_block(h, dr0w1_ref[...], dr0w2_ref[...], pv, nx)
    h = _res_block(h, dr1w1_ref[...], dr1w2_ref[...], pv, nx)
    h = jax.nn.relu(h)

    t1 = t1_ref[...]
    bt1 = bt1_ref[...]
    ye = jax.nn.relu(_mm(t1[1], h) + pv(_mm(t1[3], h)) + bt1)
    yo = jax.nn.relu(_mm(t1[2], h) + nx(_mm(t1[0], h)) + bt1)

    w2c = w2c_ref[...]
    ve = _mm(w2c, ye)
    vo = _mm(w2c, yo)
    r0 = ve[1:2] + pv(vo[3:4])
    r1 = ve[2:3] + vo[0:1]
    r2 = vo[1:2] + ve[3:4]
    r3 = vo[2:3] + nx(ve[0:1])
    r4 = jnp.concatenate([r0, r1, r2, r3], axis=0) + bt2_ref[...]

    rparts = [r4[:, g * seg:(g + 1) * seg] * stds[g] + means[g]
              for g in range(_P)]
    r4_ref[...] = jnp.concatenate(rparts, axis=-1)[None]
    idx_ref[...] = idx[None, None, :]
    hist_ref[...] = hist[None]
    sq_ref[...] = jnp.full((1, 1, 128), sq, _F32)


def _finalize_body(hist_ref, sq_ref, loss_ref, perp_ref, *, seq):
    nb = hist_ref.shape[0]
    flat_n = nb * float(seq)
    total_sq = jnp.sum(sq_ref[...][:, :, 0])
    loss = (1.0 + _CC) * total_sq / (flat_n * 64.0)
    p = jnp.sum(hist_ref[...], axis=0) / flat_n
    perp = jnp.exp(-jnp.sum(p * jnp.log(p + 1e-10)))
    loss_ref[...] = jnp.full((1, 1), loss, _F32)
    perp_ref[...] = jnp.full((1, 1), perp, _F32)


def kernel(x, enc_c1_w, enc_c1_b, enc_c2_w, enc_c2_b, enc_c3_w, enc_c3_b,
           enc_r0_w1, enc_r0_w2, enc_r1_w1, enc_r1_w2, enc_pre_w, enc_pre_b,
           emb, dec_c1_w, dec_c1_b, dec_r0_w1, dec_r0_w2, dec_r1_w1,
           dec_r1_w2, dec_t1_w, dec_t1_b, dec_t2_w, dec_t2_b):
    nb, _, ll = x.shape
    l4 = ll // 4
    ng = nb // _P
    w2s = _P * l4
    kk = emb.shape[0]

    x4 = jnp.transpose(x.reshape(ng, _P, l4, 4), (0, 3, 1, 2)).reshape(
        ng, 4, w2s)

    w1 = enc_c1_w[:, 0, :]
    w2 = jnp.transpose(enc_c2_w, (2, 0, 1))
    w3 = jnp.transpose(enc_c3_w, (2, 0, 1))
    er0w1 = jnp.transpose(enc_r0_w1, (2, 0, 1))
    er1w1 = jnp.transpose(enc_r1_w1, (2, 0, 1))
    er0w2 = enc_r0_w2[:, :, 0]
    er1w2 = enc_r1_w2[:, :, 0]
    wpre = enc_pre_w[:, :, 0]
    dw1 = jnp.transpose(dec_c1_w, (2, 0, 1))
    dr0w1 = jnp.transpose(dec_r0_w1, (2, 0, 1))
    dr1w1 = jnp.transpose(dec_r1_w1, (2, 0, 1))
    dr0w2 = dec_r0_w2[:, :, 0]
    dr1w2 = dec_r1_w2[:, :, 0]
    t1 = jnp.transpose(dec_t1_w, (2, 1, 0))
    w2c = jnp.transpose(dec_t2_w[:, 0, :], (1, 0))

    col = lambda v: v.reshape(-1, 1)
    b1, b2, b3 = col(enc_c1_b), col(enc_c2_b), col(enc_c3_b)
    bpre, db1, bt1 = col(enc_pre_b), col(dec_c1_b), col(dec_t1_b)
    bt2 = dec_t2_b.reshape(1, 1)

    full = lambda a: pl.BlockSpec(a.shape, lambda b: (0,) * a.ndim)
    r4o, idxo, histo, sqo = pl.pallas_call(
        _vqvae_body,
        grid=(ng,),
        compiler_params=pltpu.CompilerParams(
            dimension_semantics=("parallel",)),
        in_specs=[pl.BlockSpec((1, 4, w2s), lambda b: (b, 0, 0))] + [
            full(a) for a in (w1, b1, w2, b2, w3, b3, er0w1, er0w2, er1w1,
                              er1w2, wpre, bpre, emb, dw1, db1, dr0w1, dr0w2,
                              dr1w1, dr1w2, t1, bt1, w2c, bt2)],
        out_specs=[
            pl.BlockSpec((1, 4, w2s), lambda b: (b, 0, 0)),
            pl.BlockSpec((1, 1, w2s), lambda b: (b, 0, 0)),
            pl.BlockSpec((1, kk, 1), lambda b: (b, 0, 0)),
            pl.BlockSpec((1, 1, 128), lambda b: (b, 0, 0)),
        ],
        out_shape=[
            jax.ShapeDtypeStruct((ng, 4, w2s), _F32),
            jax.ShapeDtypeStruct((ng, 1, w2s), jnp.int32),
            jax.ShapeDtypeStruct((ng, kk, 1), _F32),
            jax.ShapeDtypeStruct((ng, 1, 128), _F32),
        ],
    )(x4, w1, b1, w2, b2, w3, b3, er0w1, er0w2, er1w1, er1w2, wpre, bpre,
      emb, dw1, db1, dr0w1, dr0w2, dr1w1, dr1w2, t1, bt1, w2c, bt2)

    loss2, perp2 = pl.pallas_call(
        functools.partial(_finalize_body, seq=w2s),
        out_shape=[jax.ShapeDtypeStruct((1, 1), _F32),
                   jax.ShapeDtypeStruct((1, 1), _F32)],
    )(histo, sqo)

    recon = jnp.transpose(r4o.reshape(ng, 4, _P, l4), (0, 2, 3, 1)).reshape(
        nb, 1, ll)
    idx2 = idxo.reshape(ng, _P, l4).reshape(nb, l4)
    return (recon, loss2[0, 0], idx2, perp2[0, 0])


# speedup vs baseline: 1.0411x; 1.0411x over previous
import functools

import jax
import jax.numpy as jnp
from jax.experimental import pallas as pl
from jax.experimental.pallas import tpu as pltpu

_F32 = jnp.float32
_CC = 0.25


def _mm(a, b):
    return jax.lax.dot_general(a, b, (((1,), (0,)), ((), ())),
                               preferred_element_type=_F32)


_P = 4


def _prevp(a, s):
    za = jnp.zeros_like(a[..., :1])
    parts = []
    for g in range(0, a.shape[-1], s):
        parts += [za, a[..., g:g + s - 1]]
    return jnp.concatenate(parts, axis=-1)


def _nextp(a, s):
    za = jnp.zeros_like(a[..., :1])
    parts = []
    for g in range(0, a.shape[-1], s):
        parts += [a[..., g + 1:g + s], za]
    return jnp.concatenate(parts, axis=-1)


def _res_block(x, w1, w2, pv, nx):
    a = jax.nn.relu(x)
    r = pv(_mm(w1[0], a)) + _mm(w1[1], a) + nx(_mm(w1[2], a))
    r = jax.nn.relu(r)
    return x + _mm(w2, r)


def _vqvae_body(x4_ref, w1_ref, b1_ref, w2_ref, b2_ref, w3_ref, b3_ref,
                er0w1_ref, er0w2_ref, er1w1_ref, er1w2_ref,
                wpre_ref, bpre_ref, emb_ref,
                dw1_ref, db1_ref, dr0w1_ref, dr0w2_ref, dr1w1_ref, dr1w2_ref,
                t1_ref, bt1_ref, w2c_ref, bt2_ref,
                r4_ref, idx_ref, hist_ref, sq_ref):
    xr = x4_ref[...]
    seg = xr.shape[1]
    xp = jnp.transpose(xr, (0, 2, 1))
    pv = lambda a: _prevp(a, seg)
    nx = lambda a: _nextp(a, seg)

    means, stds, xparts = [], [], []
    for g in range(_P):
        xg = xp[g]
        mg = jnp.mean(xg)
        sg = jnp.sqrt(jnp.mean((xg - mg) ** 2) + 1e-5)
        means.append(mg)
        stds.append(sg)
        xparts.append((xg - mg) / sg)
    xn = jnp.concatenate(xparts, axis=-1)

    s_e = jnp.concatenate([pv(xn[3:4]), xn[0:3]], axis=0)
    s_o = jnp.concatenate([xn[1:4], nx(xn[0:1])], axis=0)
    b1 = b1_ref[...]
    y1e = jax.nn.relu(_mm(w1_ref[...], s_e) + b1)
    y1o = jax.nn.relu(_mm(w1_ref[...], s_o) + b1)

    w2 = w2_ref[...]
    y2 = (_mm(w2[0], pv(y1o)) + _mm(w2[1], y1e) +
          _mm(w2[2], y1o) + _mm(w2[3], nx(y1e)) + b2_ref[...])
    y2 = jax.nn.relu(y2)

    w3 = w3_ref[...]
    h = (_mm(w3[0], pv(y2)) + _mm(w3[1], y2) + _mm(w3[2], nx(y2))
         + b3_ref[...])

    h = _res_block(h, er0w1_ref[...], er0w2_ref[...], pv, nx)
    h = _res_block(h, er1w1_ref[...], er1w2_ref[...], pv, nx)
    h = jax.nn.relu(h)

    z = _mm(wpre_ref[...], h) + bpre_ref[...]

    emb = emb_ref[...]
    kk = emb.shape[0]
    lf = z.shape[1]
    esq = jnp.sum(emb * emb, axis=1, keepdims=True)
    zsq = jnp.sum(z * z, axis=0, keepdims=True)
    scores = _mm(emb, z)
    d2 = zsq - 2.0 * scores + esq
    dmin = jnp.min(d2, axis=0, keepdims=True)
    kio = jax.lax.broadcasted_iota(jnp.int32, (kk, lf), 0)
    idx = jnp.min(jnp.where(d2 == dmin, kio, kk), axis=0)
    eq = (kio == idx[None, :]).astype(_F32)
    q = jax.lax.dot_general(emb, eq, (((0,), (0,)), ((), ())),
                            preferred_element_type=_F32)
    hist = jnp.sum(eq, axis=1, keepdims=True)
    sq = jnp.sum((q - z) ** 2)

    dw1 = dw1_ref[...]
    h = (_mm(dw1[0], pv(q)) + _mm(dw1[1], q) + _mm(dw1[2], nx(q))
         + db1_ref[...])

    h = _res_block(h, dr0w1_ref[...], dr0w2_ref[...], pv, nx)
    h = _res_block(h, dr1w1_ref[...], dr1w2_ref[...], pv, nx)
    h = jax.nn.relu(h)

    t1 = t1_ref[...]
    bt1 = bt1_ref[...]
    ye = jax.nn.relu(_mm(t1[1], h) + pv(_mm(t1[3], h)) + bt1)
    yo = jax.nn.relu(_mm(t1[2], h) + nx(_mm(t1[0], h)) + bt1)

    w2c = w2c_ref[...]
    ve = _mm(w2c, ye)
    vo = _mm(w2c, yo)
    r0 = ve[1:2] + pv(vo[3:4])
    r1 = ve[2:3] + vo[0:1]
    r2 = vo[1:2] + ve[3:4]
    r3 = vo[2:3] + nx(ve[0:1])
    r4 = jnp.concatenate([r0, r1, r2, r3], axis=0) + bt2_ref[...]

    rparts = [r4[:, g * seg:(g + 1) * seg] * stds[g] + means[g]
              for g in range(_P)]
    r4_ref[...] = jnp.stack(
        [jnp.transpose(rg, (1, 0)) for rg in rparts], axis=0)
    idx_ref[...] = idx[None, None, :]
    hist_ref[...] = hist[None]
    sq_ref[...] = jnp.full((1, 1, 128), sq, _F32)


def _finalize_body(hist_ref, sq_ref, loss_ref, perp_ref, *, seq):
    nb = hist_ref.shape[0]
    flat_n = nb * float(seq)
    total_sq = jnp.sum(sq_ref[...][:, :, 0])
    loss = (1.0 + _CC) * total_sq / (flat_n * 64.0)
    p = jnp.sum(hist_ref[...], axis=0) / flat_n
    perp = jnp.exp(-jnp.sum(p * jnp.log(p + 1e-10)))
    loss_ref[...] = jnp.full((1, 1), loss, _F32)
    perp_ref[...] = jnp.full((1, 1), perp, _F32)


def kernel(x, enc_c1_w, enc_c1_b, enc_c2_w, enc_c2_b, enc_c3_w, enc_c3_b,
           enc_r0_w1, enc_r0_w2, enc_r1_w1, enc_r1_w2, enc_pre_w, enc_pre_b,
           emb, dec_c1_w, dec_c1_b, dec_r0_w1, dec_r0_w2, dec_r1_w1,
           dec_r1_w2, dec_t1_w, dec_t1_b, dec_t2_w, dec_t2_b):
    nb, _, ll = x.shape
    l4 = ll // 4
    ng = nb // _P
    w2s = _P * l4
    kk = emb.shape[0]

    x3 = x.reshape(nb, l4, 4)

    w1 = enc_c1_w[:, 0, :]
    w2 = jnp.transpose(enc_c2_w, (2, 0, 1))
    w3 = jnp.transpose(enc_c3_w, (2, 0, 1))
    er0w1 = jnp.transpose(enc_r0_w1, (2, 0, 1))
    er1w1 = jnp.transpose(enc_r1_w1, (2, 0, 1))
    er0w2 = enc_r0_w2[:, :, 0]
    er1w2 = enc_r1_w2[:, :, 0]
    wpre = enc_pre_w[:, :, 0]
    dw1 = jnp.transpose(dec_c1_w, (2, 0, 1))
    dr0w1 = jnp.transpose(dec_r0_w1, (2, 0, 1))
    dr1w1 = jnp.transpose(dec_r1_w1, (2, 0, 1))
    dr0w2 = dec_r0_w2[:, :, 0]
    dr1w2 = dec_r1_w2[:, :, 0]
    t1 = jnp.transpose(dec_t1_w, (2, 1, 0))
    w2c = jnp.transpose(dec_t2_w[:, 0, :], (1, 0))

    col = lambda v: v.reshape(-1, 1)
    b1, b2, b3 = col(enc_c1_b), col(enc_c2_b), col(enc_c3_b)
    bpre, db1, bt1 = col(enc_pre_b), col(dec_c1_b), col(dec_t1_b)
    bt2 = dec_t2_b.reshape(1, 1)

    full = lambda a: pl.BlockSpec(a.shape, lambda b: (0,) * a.ndim)
    r4o, idxo, histo, sqo = pl.pallas_call(
        _vqvae_body,
        grid=(ng,),
        compiler_params=pltpu.CompilerParams(
            dimension_semantics=("parallel",)),
        in_specs=[pl.BlockSpec((_P, l4, 4), lambda b: (b, 0, 0))] + [
            full(a) for a in (w1, b1, w2, b2, w3, b3, er0w1, er0w2, er1w1,
                              er1w2, wpre, bpre, emb, dw1, db1, dr0w1, dr0w2,
                              dr1w1, dr1w2, t1, bt1, w2c, bt2)],
        out_specs=[
            pl.BlockSpec((_P, l4, 4), lambda b: (b, 0, 0)),
            pl.BlockSpec((1, 1, w2s), lambda b: (b, 0, 0)),
            pl.BlockSpec((1, kk, 1), lambda b: (b, 0, 0)),
            pl.BlockSpec((1, 1, 128), lambda b: (b, 0, 0)),
        ],
        out_shape=[
            jax.ShapeDtypeStruct((nb, l4, 4), _F32),
            jax.ShapeDtypeStruct((ng, 1, w2s), jnp.int32),
            jax.ShapeDtypeStruct((ng, kk, 1), _F32),
            jax.ShapeDtypeStruct((ng, 1, 128), _F32),
        ],
    )(x3, w1, b1, w2, b2, w3, b3, er0w1, er0w2, er1w1, er1w2, wpre, bpre,
      emb, dw1, db1, dr0w1, dr0w2, dr1w1, dr1w2, t1, bt1, w2c, bt2)

    loss2, perp2 = pl.pallas_call(
        functools.partial(_finalize_body, seq=w2s),
        out_shape=[jax.ShapeDtypeStruct((1, 1), _F32),
                   jax.ShapeDtypeStruct((1, 1), _F32)],
    )(histo, sqo)

    recon = r4o.reshape(nb, 1, ll)
    idx2 = idxo.reshape(ng, _P, l4).reshape(nb, l4)
    return (recon, loss2[0, 0], idx2, perp2[0, 0])


# speedup vs baseline: 1.1743x; 1.1280x over previous
import functools

import jax
import jax.numpy as jnp
from jax.experimental import pallas as pl
from jax.experimental.pallas import tpu as pltpu

_F32 = jnp.float32
_CC = 0.25


def _mm(a, b):
    return jax.lax.dot_general(a, b, (((1,), (0,)), ((), ())),
                               preferred_element_type=_F32)


_P = 4


def _prevp(a, s):
    za = jnp.zeros_like(a[..., :1])
    parts = []
    for g in range(0, a.shape[-1], s):
        parts += [za, a[..., g:g + s - 1]]
    return jnp.concatenate(parts, axis=-1)


def _nextp(a, s):
    za = jnp.zeros_like(a[..., :1])
    parts = []
    for g in range(0, a.shape[-1], s):
        parts += [a[..., g + 1:g + s], za]
    return jnp.concatenate(parts, axis=-1)


def _res_block(x, w1, w2, pv, nx):
    a = jax.nn.relu(x)
    r = pv(_mm(w1[0], a)) + _mm(w1[1], a) + nx(_mm(w1[2], a))
    r = jax.nn.relu(r)
    return x + _mm(w2, r)


def _vqvae_body(x4_ref, w1_ref, b1_ref, w2_ref, b2_ref, w3_ref, b3_ref,
                er0w1_ref, er0w2_ref, er1w1_ref, er1w2_ref,
                wpre_ref, bpre_ref, emb_ref,
                dw1_ref, db1_ref, dr0w1_ref, dr0w2_ref, dr1w1_ref, dr1w2_ref,
                t1_ref, bt1_ref, w2c_ref, bt2_ref,
                r4_ref, idx_ref, hist_ref, sq_ref):
    xp = x4_ref[...]
    seg = xp.shape[-1]
    pv = lambda a: _prevp(a, seg)
    nx = lambda a: _nextp(a, seg)

    means, stds, xparts = [], [], []
    for g in range(_P):
        xg = xp[g]
        mg = jnp.mean(xg)
        sg = jnp.sqrt(jnp.mean((xg - mg) ** 2) + 1e-5)
        means.append(mg)
        stds.append(sg)
        xparts.append((xg - mg) / sg)
    xn = jnp.concatenate(xparts, axis=-1)

    s_e = jnp.concatenate([pv(xn[3:4]), xn[0:3]], axis=0)
    s_o = jnp.concatenate([xn[1:4], nx(xn[0:1])], axis=0)
    b1 = b1_ref[...]
    y1e = jax.nn.relu(_mm(w1_ref[...], s_e) + b1)
    y1o = jax.nn.relu(_mm(w1_ref[...], s_o) + b1)

    w2 = w2_ref[...]
    y2 = (_mm(w2[0], pv(y1o)) + _mm(w2[1], y1e) +
          _mm(w2[2], y1o) + _mm(w2[3], nx(y1e)) + b2_ref[...])
    y2 = jax.nn.relu(y2)

    w3 = w3_ref[...]
    h = (_mm(w3[0], pv(y2)) + _mm(w3[1], y2) + _mm(w3[2], nx(y2))
         + b3_ref[...])

    h = _res_block(h, er0w1_ref[...], er0w2_ref[...], pv, nx)
    h = _res_block(h, er1w1_ref[...], er1w2_ref[...], pv, nx)
    h = jax.nn.relu(h)

    z = _mm(wpre_ref[...], h) + bpre_ref[...]

    emb = emb_ref[...]
    kk = emb.shape[0]
    lf = z.shape[1]
    esq = jnp.sum(emb * emb, axis=1, keepdims=True)
    zsq = jnp.sum(z * z, axis=0, keepdims=True)
    scores = _mm(emb, z)
    d2 = zsq - 2.0 * scores + esq
    dmin = jnp.min(d2, axis=0, keepdims=True)
    kio = jax.lax.broadcasted_iota(jnp.int32, (kk, lf), 0)
    idx = jnp.min(jnp.where(d2 == dmin, kio, kk), axis=0)
    eq = (kio == idx[None, :]).astype(_F32)
    q = jax.lax.dot_general(emb, eq, (((0,), (0,)), ((), ())),
                            preferred_element_type=_F32)
    hist = jnp.sum(eq, axis=1, keepdims=True)
    sq = jnp.sum((q - z) ** 2)

    dw1 = dw1_ref[...]
    h = (_mm(dw1[0], pv(q)) + _mm(dw1[1], q) + _mm(dw1[2], nx(q))
         + db1_ref[...])

    h = _res_block(h, dr0w1_ref[...], dr0w2_ref[...], pv, nx)
    h = _res_block(h, dr1w1_ref[...], dr1w2_ref[...], pv, nx)
    h = jax.nn.relu(h)

    t1 = t1_ref[...]
    bt1 = bt1_ref[...]
    ye = jax.nn.relu(_mm(t1[1], h) + pv(_mm(t1[3], h)) + bt1)
    yo = jax.nn.relu(_mm(t1[2], h) + nx(_mm(t1[0], h)) + bt1)

    w2c = w2c_ref[...]
    ve = _mm(w2c, ye)
    vo = _mm(w2c, yo)
    r0 = ve[1:2] + pv(vo[3:4])
    r1 = ve[2:3] + vo[0:1]
    r2 = vo[1:2] + ve[3:4]
    r3 = vo[2:3] + nx(ve[0:1])
    r4 = jnp.concatenate([r0, r1, r2, r3], axis=0) + bt2_ref[...]

    rparts = [r4[:, g * seg:(g + 1) * seg] * stds[g] + means[g]
              for g in range(_P)]
    r4_ref[...] = jnp.stack(rparts, axis=0)
    idx_ref[...] = idx[None, None, :]
    hist_ref[...] = hist[None]
    sq_ref[...] = jnp.full((1, 1, 128), sq, _F32)


def _finalize_body(hist_ref, sq_ref, loss_ref, perp_ref, *, seq):
    nb = hist_ref.shape[0]
    flat_n = nb * float(seq)
    total_sq = jnp.sum(sq_ref[...][:, :, 0])
    loss = (1.0 + _CC) * total_sq / (flat_n * 64.0)
    p = jnp.sum(hist_ref[...], axis=0) / flat_n
    perp = jnp.exp(-jnp.sum(p * jnp.log(p + 1e-10)))
    loss_ref[...] = jnp.full((1, 1), loss, _F32)
    perp_ref[...] = jnp.full((1, 1), perp, _F32)


def kernel(x, enc_c1_w, enc_c1_b, enc_c2_w, enc_c2_b, enc_c3_w, enc_c3_b,
           enc_r0_w1, enc_r0_w2, enc_r1_w1, enc_r1_w2, enc_pre_w, enc_pre_b,
           emb, dec_c1_w, dec_c1_b, dec_r0_w1, dec_r0_w2, dec_r1_w1,
           dec_r1_w2, dec_t1_w, dec_t1_b, dec_t2_w, dec_t2_b):
    nb, _, ll = x.shape
    l4 = ll // 4
    ng = nb // _P
    w2s = _P * l4
    kk = emb.shape[0]

    x4 = jnp.transpose(x.reshape(nb, l4, 4), (0, 2, 1))

    w1 = enc_c1_w[:, 0, :]
    w2 = jnp.transpose(enc_c2_w, (2, 0, 1))
    w3 = jnp.transpose(enc_c3_w, (2, 0, 1))
    er0w1 = jnp.transpose(enc_r0_w1, (2, 0, 1))
    er1w1 = jnp.transpose(enc_r1_w1, (2, 0, 1))
    er0w2 = enc_r0_w2[:, :, 0]
    er1w2 = enc_r1_w2[:, :, 0]
    wpre = enc_pre_w[:, :, 0]
    dw1 = jnp.transpose(dec_c1_w, (2, 0, 1))
    dr0w1 = jnp.transpose(dec_r0_w1, (2, 0, 1))
    dr1w1 = jnp.transpose(dec_r1_w1, (2, 0, 1))
    dr0w2 = dec_r0_w2[:, :, 0]
    dr1w2 = dec_r1_w2[:, :, 0]
    t1 = jnp.transpose(dec_t1_w, (2, 1, 0))
    w2c = jnp.transpose(dec_t2_w[:, 0, :], (1, 0))

    col = lambda v: v.reshape(-1, 1)
    b1, b2, b3 = col(enc_c1_b), col(enc_c2_b), col(enc_c3_b)
    bpre, db1, bt1 = col(enc_pre_b), col(dec_c1_b), col(dec_t1_b)
    bt2 = dec_t2_b.reshape(1, 1)

    full = lambda a: pl.BlockSpec(a.shape, lambda b: (0,) * a.ndim)
    r4o, idxo, histo, sqo = pl.pallas_call(
        _vqvae_body,
        grid=(ng,),
        compiler_params=pltpu.CompilerParams(
            dimension_semantics=("parallel",)),
        in_specs=[pl.BlockSpec((_P, 4, l4), lambda b: (b, 0, 0))] + [
            full(a) for a in (w1, b1, w2, b2, w3, b3, er0w1, er0w2, er1w1,
                              er1w2, wpre, bpre, emb, dw1, db1, dr0w1, dr0w2,
                              dr1w1, dr1w2, t1, bt1, w2c, bt2)],
        out_specs=[
            pl.BlockSpec((_P, 4, l4), lambda b: (b, 0, 0)),
            pl.BlockSpec((1, 1, w2s), lambda b: (b, 0, 0)),
            pl.BlockSpec((1, kk, 1), lambda b: (b, 0, 0)),
            pl.BlockSpec((1, 1, 128), lambda b: (b, 0, 0)),
        ],
        out_shape=[
            jax.ShapeDtypeStruct((nb, 4, l4), _F32),
            jax.ShapeDtypeStruct((ng, 1, w2s), jnp.int32),
            jax.ShapeDtypeStruct((ng, kk, 1), _F32),
            jax.ShapeDtypeStruct((ng, 1, 128), _F32),
        ],
    )(x4, w1, b1, w2, b2, w3, b3, er0w1, er0w2, er1w1, er1w2, wpre, bpre,
      emb, dw1, db1, dr0w1, dr0w2, dr1w1, dr1w2, t1, bt1, w2c, bt2)

    loss2, perp2 = pl.pallas_call(
        functools.partial(_finalize_body, seq=w2s),
        out_shape=[jax.ShapeDtypeStruct((1, 1), _F32),
                   jax.ShapeDtypeStruct((1, 1), _F32)],
    )(histo, sqo)

    recon = jnp.transpose(r4o, (0, 2, 1)).reshape(nb, 1, ll)
    idx2 = idxo.reshape(ng, _P, l4).reshape(nb, l4)
    return (recon, loss2[0, 0], idx2, perp2[0, 0])
